# Initial kernel scaffold; baseline (speedup 1.0000x reference)
import jax
import jax.numpy as jnp
from jax.experimental import pallas as pl


def kernel(x, edge_index, W1, b1, W2, b2, W3, b3, W4, b4, Wout, bout):
    raise NotImplementedError("write your pallas kernel here")


# speedup vs baseline: 3.4916x; 3.4916x over previous
import functools

import jax
import jax.numpy as jnp
from jax.experimental import pallas as pl
from jax.experimental.pallas import tpu as pltpu

N = 10000
NP = 10240


def _mm_kernel(h_ref, w_ref, o_ref):
    o_ref[...] = jnp.dot(h_ref[...], w_ref[...],
                         preferred_element_type=jnp.float32)


def _matmul(h, w, bm=2048):
    m, k = h.shape
    n = w.shape[1]
    return pl.pallas_call(
        _mm_kernel,
        grid=(m // bm,),
        in_specs=[
            pl.BlockSpec((bm, k), lambda i: (i, 0)),
            pl.BlockSpec((k, n), lambda i: (0, 0)),
        ],
        out_specs=pl.BlockSpec((bm, n), lambda i: (i, 0)),
        out_shape=jax.ShapeDtypeStruct((m, n), jnp.float32),
        compiler_params=pltpu.CompilerParams(
            dimension_semantics=("parallel",)),
    )(h, w)


def _agg_kernel(a_ref, hw_ref, b_ref, o_ref):
    k = pl.program_id(1)

    @pl.when(k == 0)
    def _init():
        o_ref[...] = jnp.zeros_like(o_ref)

    o_ref[...] += jnp.dot(a_ref[...], hw_ref[...],
                          preferred_element_type=jnp.float32)

    @pl.when(k == pl.num_programs(1) - 1)
    def _fin():
        o_ref[...] = jnp.maximum(o_ref[...] + b_ref[...], 0.0)


def _aggregate(a, hw, b, bm=1024, bk=2048):
    m = a.shape[0]
    n = hw.shape[1]
    return pl.pallas_call(
        _agg_kernel,
        grid=(m // bm, m // bk),
        in_specs=[
            pl.BlockSpec((bm, bk), lambda i, k: (i, k)),
            pl.BlockSpec((bk, n), lambda i, k: (k, 0)),
            pl.BlockSpec((1, n), lambda i, k: (0, 0)),
        ],
        out_specs=pl.BlockSpec((bm, n), lambda i, k: (i, 0)),
        out_shape=jax.ShapeDtypeStruct((m, n), jnp.float32),
        compiler_params=pltpu.CompilerParams(
            dimension_semantics=("parallel", "arbitrary")),
    )(a, hw, b)


def _outer_kernel(vr_ref, vt_ref, o_ref):
    o_ref[...] = jnp.maximum(vr_ref[...] * vt_ref[...], 0.0)


def _outer_relu(v, bm=400):
    vt = v.reshape(1, N)
    return pl.pallas_call(
        _outer_kernel,
        grid=(N // bm,),
        in_specs=[
            pl.BlockSpec((bm, 1), lambda i: (i, 0)),
            pl.BlockSpec((1, N), lambda i: (0, 0)),
        ],
        out_specs=pl.BlockSpec((bm, N), lambda i: (i, 0)),
        out_shape=jax.ShapeDtypeStruct((N, N), jnp.float32),
        compiler_params=pltpu.CompilerParams(
            dimension_semantics=("parallel",)),
    )(v, vt)


def kernel(x, edge_index, W1, b1, W2, b2, W3, b3, W4, b4, Wout, bout):
    src = edge_index[0].astype(jnp.int32)
    dst = edge_index[1].astype(jnp.int32)

    ones = jnp.ones(src.shape, jnp.float32)
    deg = jnp.ones((N,), jnp.float32).at[dst].add(ones)
    dinv = jax.lax.rsqrt(deg)
    norm = dinv[src] * dinv[dst]

    a = jnp.zeros((NP, NP), jnp.float32)
    a = a.at[dst, src].add(norm)
    diag = jnp.arange(N, dtype=jnp.int32)
    a = a.at[diag, diag].add(dinv * dinv)

    h = jnp.pad(x, ((0, NP - N), (0, 0)))
    for w, b in ((W1, b1), (W2, b2), (W3, b3), (W4, b4)):
        hw = _matmul(h, w)
        h = _aggregate(a, hw, b.reshape(1, -1))

    v = _matmul(h, Wout)
    v = (v + bout)[:N]
    return _outer_relu(v)
